# Initial kernel scaffold; baseline (speedup 1.0000x reference)
import jax
import jax.numpy as jnp
from jax.experimental import pallas as pl


def kernel(x, edge_index, batch, W0, b0, W1, b1, W2, b2, W_eos, b_eos):
    raise NotImplementedError("write your pallas kernel here")


# speedup vs baseline: 13.8273x; 13.8273x over previous
import functools

import jax
import jax.numpy as jnp
from jax import lax
from jax.experimental import pallas as pl
from jax.experimental.pallas import tpu as pltpu
from jax.experimental.pallas import tpu_sc as plsc

N = 10000
E = 320000
D = 128
H = 64
G = 8
NEG_SLOPE = 0.01

P = 10240
EB = 128
TILES = 32
NBLK = 80
EP = TILES * NBLK * EB
ROWS_PER_TILE = P // 16
W1W = 16
TCB = 512
TCG = P // TCB

_mesh = plsc.VectorSubcoreMesh(core_axis_name="c", subcore_axis_name="s")
_f32 = jnp.float32
_params = pltpu.CompilerParams(use_tc_tiling_on_sc=False)


def _zero_phase(s, zsrcs_dsts):
    for zsrc, dst in zsrcs_dsts:
        pltpu.sync_copy(zsrc, dst.at[pl.ds(s * ROWS_PER_TILE, ROWS_PER_TILE)])
    plsc.subcore_barrier()


def _copy_out(c, s, pairs):
    plsc.subcore_barrier()
    for sh, out in pairs:
        sl = pl.ds(s * ROWS_PER_TILE, ROWS_PER_TILE)
        pltpu.sync_copy(sh.at[sl], out.at[c, sl])


@functools.partial(
    pl.kernel,
    mesh=_mesh,
    compiler_params=_params,
    out_type=jax.ShapeDtypeStruct((2, P, W1W), _f32),
    scratch_types=[
        pltpu.VMEM((NBLK, EB), jnp.int32),
        pltpu.VMEM((EB, W1W), _f32),
        pltpu.VMEM_SHARED((P, W1W), _f32),
    ],
)
def _sc_deg(dstm, ones_hbm, zeros16, deg_out, didx, onesv, accd):
    c = lax.axis_index("c")
    s = lax.axis_index("s")
    wid = s * 2 + c
    pltpu.sync_copy(ones_hbm, onesv)
    pltpu.sync_copy(dstm.at[pl.ds(wid * NBLK, NBLK)], didx)
    _zero_phase(s, [(zeros16, accd)])

    @pl.loop(0, NBLK)
    def _(j):
        pltpu.sync_copy(onesv, accd.at[didx.at[j]], add=True)

    _copy_out(c, s, [(accd, deg_out)])


@functools.partial(
    pl.kernel,
    mesh=_mesh,
    compiler_params=_params,
    out_type=[
        jax.ShapeDtypeStruct((2, P, H), _f32),
        jax.ShapeDtypeStruct((2, P, W1W), _f32),
    ],
    scratch_types=[
        pltpu.VMEM((NBLK, EB), jnp.int32),
        pltpu.VMEM((NBLK, EB), jnp.int32),
        pltpu.VMEM((EB, H), _f32),
        pltpu.VMEM((EB, W1W), _f32),
        pltpu.VMEM_SHARED((P, H), _f32),
        pltpu.VMEM_SHARED((P, W1W), _f32),
        pltpu.SemaphoreType.DMA,
        pltpu.SemaphoreType.DMA,
    ],
)
def _sc_prop64_ride(srcm, dstm, tab, dtab, zeros64, zeros16,
                    acc_out, accd_out,
                    sidx, didx, rows, vals, acc, accd, sem, sem2):
    c = lax.axis_index("c")
    s = lax.axis_index("s")
    wid = s * 2 + c
    pltpu.sync_copy(srcm.at[pl.ds(wid * NBLK, NBLK)], sidx)
    pltpu.sync_copy(dstm.at[pl.ds(wid * NBLK, NBLK)], didx)
    _zero_phase(s, [(zeros64, acc), (zeros16, accd)])

    @pl.loop(0, NBLK)
    def _(j):
        g1 = pltpu.async_copy(tab.at[sidx.at[j]], rows, sem)
        g2 = pltpu.async_copy(dtab.at[sidx.at[j]], vals, sem2)
        g1.wait()
        pltpu.sync_copy(rows, acc.at[didx.at[j]], add=True)
        g2.wait()
        pltpu.sync_copy(vals, accd.at[didx.at[j]], add=True)

    _copy_out(c, s, [(acc, acc_out), (accd, accd_out)])


@functools.partial(
    pl.kernel,
    mesh=_mesh,
    compiler_params=_params,
    out_type=jax.ShapeDtypeStruct((2, P, H), _f32),
    scratch_types=[
        pltpu.VMEM((NBLK, EB), jnp.int32),
        pltpu.VMEM((NBLK, EB), jnp.int32),
        pltpu.VMEM((EB, H), _f32),
        pltpu.VMEM_SHARED((P, H), _f32),
        pltpu.SemaphoreType.DMA,
    ],
)
def _sc_prop64(srcm, dstm, tab, zeros64, acc_out, sidx, didx, rows, acc, sem):
    c = lax.axis_index("c")
    s = lax.axis_index("s")
    wid = s * 2 + c
    pltpu.sync_copy(srcm.at[pl.ds(wid * NBLK, NBLK)], sidx)
    pltpu.sync_copy(dstm.at[pl.ds(wid * NBLK, NBLK)], didx)
    _zero_phase(s, [(zeros64, acc)])

    @pl.loop(0, NBLK)
    def _(j):
        pltpu.async_copy(tab.at[sidx.at[j]], rows, sem).wait()
        pltpu.sync_copy(rows, acc.at[didx.at[j]], add=True)

    _copy_out(c, s, [(acc, acc_out)])


@functools.partial(
    pl.kernel,
    mesh=_mesh,
    compiler_params=_params,
    out_type=jax.ShapeDtypeStruct((2, P, W1W), _f32),
    scratch_types=[
        pltpu.VMEM((NBLK, EB), jnp.int32),
        pltpu.VMEM((NBLK, EB), jnp.int32),
        pltpu.VMEM((EB, W1W), _f32),
        pltpu.VMEM_SHARED((P, W1W), _f32),
        pltpu.SemaphoreType.DMA,
    ],
)
def _sc_prop16(srcm, dstm, tab, zeros16, acc_out, sidx, didx, vals, accd, sem):
    c = lax.axis_index("c")
    s = lax.axis_index("s")
    wid = s * 2 + c
    pltpu.sync_copy(srcm.at[pl.ds(wid * NBLK, NBLK)], sidx)
    pltpu.sync_copy(dstm.at[pl.ds(wid * NBLK, NBLK)], didx)
    _zero_phase(s, [(zeros16, accd)])

    @pl.loop(0, NBLK)
    def _(j):
        pltpu.async_copy(tab.at[sidx.at[j]], vals, sem).wait()
        pltpu.sync_copy(vals, accd.at[didx.at[j]], add=True)

    _copy_out(c, s, [(accd, acc_out)])


def _tc_z0_body(x_ref, w0_ref, w1_ref, b0_ref, z0_ref, bw1_ref):
    u = jnp.dot(x_ref[...], w0_ref[...], preferred_element_type=_f32)
    z0_ref[...] = jnp.dot(u, w1_ref[...], preferred_element_type=_f32)

    @pl.when(pl.program_id(0) == 0)
    def _():
        bw1_ref[...] = jnp.dot(b0_ref[...], w1_ref[...],
                               preferred_element_type=_f32)


_tc_z0 = pl.pallas_call(
    _tc_z0_body,
    grid=(TCG,),
    in_specs=[
        pl.BlockSpec((TCB, D), lambda i: (i, 0)),
        pl.BlockSpec((D, D), lambda i: (0, 0)),
        pl.BlockSpec((D, H), lambda i: (0, 0)),
        pl.BlockSpec((1, D), lambda i: (0, 0)),
    ],
    out_specs=[
        pl.BlockSpec((TCB, H), lambda i: (i, 0)),
        pl.BlockSpec((1, H), lambda i: (0, 0)),
    ],
    out_shape=[
        jax.ShapeDtypeStruct((P, H), _f32),
        jax.ShapeDtypeStruct((1, H), _f32),
    ],
)


def _tc_prep_body(degp_ref, z0_ref, dinv_ref, dinv16_ref, z0n_ref):
    deg = degp_ref[0][:, 0:1] + degp_ref[1][:, 0:1] + 1.0
    dv = 1.0 / jnp.sqrt(deg)
    dinv_ref[...] = dv
    dinv16_ref[...] = dv * jnp.ones((1, W1W), _f32)
    z0n_ref[...] = dv * z0_ref[...]


_tc_prep = pl.pallas_call(
    _tc_prep_body,
    grid=(TCG,),
    in_specs=[
        pl.BlockSpec((2, TCB, W1W), lambda i: (0, i, 0)),
        pl.BlockSpec((TCB, H), lambda i: (i, 0)),
    ],
    out_specs=[
        pl.BlockSpec((TCB, 1), lambda i: (i, 0)),
        pl.BlockSpec((TCB, W1W), lambda i: (i, 0)),
        pl.BlockSpec((TCB, H), lambda i: (i, 0)),
    ],
    out_shape=[
        jax.ShapeDtypeStruct((P, 1), _f32),
        jax.ShapeDtypeStruct((P, W1W), _f32),
        jax.ShapeDtypeStruct((P, H), _f32),
    ],
)


def _tc_t1n_body(acc1_ref, z0n_ref, dinv_ref, t1n_ref):
    dv = dinv_ref[...]
    t1n_ref[...] = dv * dv * (acc1_ref[0] + acc1_ref[1] + z0n_ref[...])


_tc_t1n = pl.pallas_call(
    _tc_t1n_body,
    grid=(TCG,),
    in_specs=[
        pl.BlockSpec((2, TCB, H), lambda i: (0, i, 0)),
        pl.BlockSpec((TCB, H), lambda i: (i, 0)),
        pl.BlockSpec((TCB, 1), lambda i: (i, 0)),
    ],
    out_specs=pl.BlockSpec((TCB, H), lambda i: (i, 0)),
    out_shape=jax.ShapeDtypeStruct((P, H), _f32),
)


def _tc_h1_body(acc2_ref, t1n_ref, dinv_ref, sdinv_ref, bw1_ref, b1_ref,
                w2_ref, batch_ref, z2n16_ref, pool_ref, cnt_ref):
    i = pl.program_id(0)
    dv = dinv_ref[...]
    q = dv * (sdinv_ref[0][:, 0:1] + sdinv_ref[1][:, 0:1] + dv)
    hpre = (dv * (acc2_ref[0] + acc2_ref[1] + t1n_ref[...])
            + q * bw1_ref[...] + b1_ref[...])
    h1 = jnp.where(hpre > 0, hpre, NEG_SLOPE * hpre)
    z2 = jnp.sum(h1 * w2_ref[...], axis=1, keepdims=True)
    z2n16_ref[...] = (dv * z2) * jnp.ones((1, W1W), _f32)
    b = batch_ref[...]
    gids = lax.broadcasted_iota(jnp.int32, (TCB, G), 1)
    onehot = (b == gids).astype(_f32)
    pool_blk = lax.dot_general(onehot, h1, (((0,), (0,)), ((), ())),
                               preferred_element_type=_f32)
    cnt_blk = lax.dot_general(onehot, jnp.ones((TCB, 1), _f32),
                              (((0,), (0,)), ((), ())),
                              preferred_element_type=_f32)

    @pl.when(i == 0)
    def _():
        pool_ref[...] = pool_blk
        cnt_ref[...] = cnt_blk

    @pl.when(i > 0)
    def _():
        pool_ref[...] += pool_blk
        cnt_ref[...] += cnt_blk


_tc_h1 = pl.pallas_call(
    _tc_h1_body,
    grid=(TCG,),
    in_specs=[
        pl.BlockSpec((2, TCB, H), lambda i: (0, i, 0)),
        pl.BlockSpec((TCB, H), lambda i: (i, 0)),
        pl.BlockSpec((TCB, 1), lambda i: (i, 0)),
        pl.BlockSpec((2, TCB, W1W), lambda i: (0, i, 0)),
        pl.BlockSpec((1, H), lambda i: (0, 0)),
        pl.BlockSpec((1, H), lambda i: (0, 0)),
        pl.BlockSpec((1, H), lambda i: (0, 0)),
        pl.BlockSpec((TCB, 1), lambda i: (i, 0)),
    ],
    out_specs=[
        pl.BlockSpec((TCB, W1W), lambda i: (i, 0)),
        pl.BlockSpec((G, H), lambda i: (0, 0)),
        pl.BlockSpec((G, 1), lambda i: (0, 0)),
    ],
    out_shape=[
        jax.ShapeDtypeStruct((P, W1W), _f32),
        jax.ShapeDtypeStruct((G, H), _f32),
        jax.ShapeDtypeStruct((G, 1), _f32),
    ],
)


def _tc_final_body(accz_ref, z2n_ref, dinv_ref, b2_ref, pool_ref, cnt_ref,
                   weos_ref, beos_ref, yl_ref, ye_ref):
    dv = dinv_ref[...]
    yl_ref[...] = (dv * (accz_ref[0][:, 0:1] + accz_ref[1][:, 0:1]
                         + z2n_ref[...][:, 0:1]) + b2_ref[...])

    @pl.when(pl.program_id(0) == 0)
    def _():
        xp = pool_ref[...] / jnp.maximum(cnt_ref[...], 1.0)
        ye_ref[...] = (jnp.sum(xp * weos_ref[...], axis=1, keepdims=True)
                       + beos_ref[...])


_tc_final = pl.pallas_call(
    _tc_final_body,
    grid=(TCG,),
    in_specs=[
        pl.BlockSpec((2, TCB, W1W), lambda i: (0, i, 0)),
        pl.BlockSpec((TCB, W1W), lambda i: (i, 0)),
        pl.BlockSpec((TCB, 1), lambda i: (i, 0)),
        pl.BlockSpec((1, 1), lambda i: (0, 0)),
        pl.BlockSpec((G, H), lambda i: (0, 0)),
        pl.BlockSpec((G, 1), lambda i: (0, 0)),
        pl.BlockSpec((1, H), lambda i: (0, 0)),
        pl.BlockSpec((1, 1), lambda i: (0, 0)),
    ],
    out_specs=[
        pl.BlockSpec((TCB, 1), lambda i: (i, 0)),
        pl.BlockSpec((G, 1), lambda i: (0, 0)),
    ],
    out_shape=[
        jax.ShapeDtypeStruct((P, 1), _f32),
        jax.ShapeDtypeStruct((G, 1), _f32),
    ],
)


def kernel(x, edge_index, batch, W0, b0, W1, b1, W2, b2, W_eos, b_eos):
    src = edge_index[0].astype(jnp.int32)
    dst = edge_index[1].astype(jnp.int32)
    pad = jnp.full((EP - E,), N, jnp.int32)
    srcm = jnp.concatenate([src, pad]).reshape(EP // EB, EB)
    dstm = jnp.concatenate([dst, pad]).reshape(EP // EB, EB)

    x_pad = jnp.pad(x, ((0, P - N), (0, 0)))
    batchp = jnp.pad(batch.astype(jnp.int32), (0, P - N),
                     constant_values=G).reshape(P, 1)

    zeros64 = jnp.zeros((ROWS_PER_TILE, H), _f32)
    zeros16 = jnp.zeros((ROWS_PER_TILE, W1W), _f32)
    ones_eb = jnp.ones((EB, W1W), _f32)

    b0r = b0.reshape(1, D)
    b1r = b1.reshape(1, H)
    w2r = W2.reshape(1, H)
    b2r = b2.reshape(1, 1)
    weosr = W_eos.reshape(1, H)
    beosr = b_eos.reshape(1, 1)

    z0, bw1 = _tc_z0(x_pad, W0, W1, b0r)
    degp = _sc_deg(dstm, ones_eb, zeros16)
    dinv, dinv16, z0n = _tc_prep(degp, z0)
    acc1, sdinv = _sc_prop64_ride(srcm, dstm, z0n, dinv16, zeros64, zeros16)
    t1n = _tc_t1n(acc1, z0n, dinv)
    acc2 = _sc_prop64(srcm, dstm, t1n, zeros64)
    z2n16, pool, counts = _tc_h1(acc2, t1n, dinv, sdinv, bw1, b1r, w2r, batchp)
    accz = _sc_prop16(srcm, dstm, z2n16, zeros16)
    y_leafp, y_eosp = _tc_final(accz, z2n16, dinv, b2r, pool, counts,
                                weosr, beosr)
    return y_leafp[:N, 0], y_eosp[:, 0]


# speedup vs baseline: 17.4145x; 1.2594x over previous
import functools

import jax
import jax.numpy as jnp
from jax import lax
from jax.experimental import pallas as pl
from jax.experimental.pallas import tpu as pltpu
from jax.experimental.pallas import tpu_sc as plsc

N = 10000
E = 320000
D = 128
H = 64
G = 8
NEG_SLOPE = 0.01

P = 10240
EB = 128
TILES = 32
NBLK = 80
EP = TILES * NBLK * EB
ROWS_PER_TILE = P // 16
W1W = 16
TCB = 512
TCG = P // TCB

_mesh = plsc.VectorSubcoreMesh(core_axis_name="c", subcore_axis_name="s")
_f32 = jnp.float32
_params = pltpu.CompilerParams(use_tc_tiling_on_sc=False)


def _zero_phase(s, zsrcs_dsts):
    for zsrc, dst in zsrcs_dsts:
        pltpu.sync_copy(zsrc, dst.at[pl.ds(s * ROWS_PER_TILE, ROWS_PER_TILE)])
    plsc.subcore_barrier()


def _copy_out(c, s, pairs):
    plsc.subcore_barrier()
    for sh, out in pairs:
        sl = pl.ds(s * ROWS_PER_TILE, ROWS_PER_TILE)
        pltpu.sync_copy(sh.at[sl], out.at[c, sl])


@functools.partial(
    pl.kernel,
    mesh=_mesh,
    compiler_params=_params,
    out_type=jax.ShapeDtypeStruct((2, P, W1W), _f32),
    scratch_types=[
        pltpu.VMEM((NBLK, EB), jnp.int32),
        pltpu.VMEM((EB, W1W), _f32),
        pltpu.VMEM_SHARED((P, W1W), _f32),
    ],
)
def _sc_deg(dstm, ones_hbm, zeros16, deg_out, didx, onesv, accd):
    c = lax.axis_index("c")
    s = lax.axis_index("s")
    wid = s * 2 + c
    pltpu.sync_copy(ones_hbm, onesv)
    pltpu.sync_copy(dstm.at[pl.ds(wid * NBLK, NBLK)], didx)
    _zero_phase(s, [(zeros16, accd)])

    @pl.loop(0, NBLK)
    def _(j):
        pltpu.sync_copy(onesv, accd.at[didx.at[j]], add=True)

    _copy_out(c, s, [(accd, deg_out)])


def _prop_loop(tab, sidx, didx, rows0, rows1, acc, sem0, sem1, nblk):
    pltpu.async_copy(tab.at[sidx.at[0]], rows0, sem0)

    @pl.loop(0, nblk, step=2)
    def _(j):
        g1 = pltpu.async_copy(tab.at[sidx.at[j + 1]], rows1, sem1)
        pltpu.make_async_copy(tab.at[sidx.at[j]], rows0, sem0).wait()
        pltpu.sync_copy(rows0, acc.at[didx.at[j]], add=True)

        @pl.when(j + 2 < nblk)
        def _():
            pltpu.async_copy(tab.at[sidx.at[j + 2]], rows0, sem0)

        g1.wait()
        pltpu.sync_copy(rows1, acc.at[didx.at[j + 1]], add=True)


NBLK2 = 2 * NBLK


@functools.partial(
    pl.kernel,
    mesh=_mesh,
    compiler_params=_params,
    out_type=jax.ShapeDtypeStruct((2, P, H), _f32),
    scratch_types=[
        pltpu.VMEM((NBLK2, EB), jnp.int32),
        pltpu.VMEM((NBLK2, EB), jnp.int32),
        pltpu.VMEM((EB, H), _f32),
        pltpu.VMEM((EB, H), _f32),
        pltpu.VMEM_SHARED((P, H), _f32),
        pltpu.SemaphoreType.DMA,
        pltpu.SemaphoreType.DMA,
    ],
)
def _sc_prop128(srcm, dstm, tab_lo, tab_hi, zeros, acc_out,
                sidx, didx, rows0, rows1, acc, sem0, sem1):
    c = lax.axis_index("c")
    s = lax.axis_index("s")
    pltpu.sync_copy(srcm.at[pl.ds(s * NBLK2, NBLK2)], sidx)
    pltpu.sync_copy(dstm.at[pl.ds(s * NBLK2, NBLK2)], didx)
    _zero_phase(s, [(zeros, acc)])

    @pl.when(c == 0)
    def _():
        _prop_loop(tab_lo, sidx, didx, rows0, rows1, acc, sem0, sem1, NBLK2)

    @pl.when(c == 1)
    def _():
        _prop_loop(tab_hi, sidx, didx, rows0, rows1, acc, sem0, sem1, NBLK2)

    _copy_out(c, s, [(acc, acc_out)])


def _make_prop(width):
    @functools.partial(
        pl.kernel,
        mesh=_mesh,
        compiler_params=_params,
        out_type=jax.ShapeDtypeStruct((2, P, width), _f32),
        scratch_types=[
            pltpu.VMEM((NBLK, EB), jnp.int32),
            pltpu.VMEM((NBLK, EB), jnp.int32),
            pltpu.VMEM((EB, width), _f32),
            pltpu.VMEM((EB, width), _f32),
            pltpu.VMEM_SHARED((P, width), _f32),
            pltpu.SemaphoreType.DMA,
            pltpu.SemaphoreType.DMA,
        ],
    )
    def _prop(srcm, dstm, tab, zeros, acc_out,
              sidx, didx, rows0, rows1, acc, sem0, sem1):
        c = lax.axis_index("c")
        s = lax.axis_index("s")
        wid = s * 2 + c
        pltpu.sync_copy(srcm.at[pl.ds(wid * NBLK, NBLK)], sidx)
        pltpu.sync_copy(dstm.at[pl.ds(wid * NBLK, NBLK)], didx)
        _zero_phase(s, [(zeros, acc)])
        _prop_loop(tab, sidx, didx, rows0, rows1, acc, sem0, sem1, NBLK)
        _copy_out(c, s, [(acc, acc_out)])

    return _prop


_sc_prop64 = _make_prop(H)
_sc_prop16 = _make_prop(W1W)


def _tc_u0_body(x_ref, w0_ref, u0_ref):
    u0_ref[...] = jnp.dot(x_ref[...], w0_ref[...], preferred_element_type=_f32)


_tc_u0 = pl.pallas_call(
    _tc_u0_body,
    grid=(TCG,),
    in_specs=[
        pl.BlockSpec((TCB, D), lambda i: (i, 0)),
        pl.BlockSpec((D, D), lambda i: (0, 0)),
    ],
    out_specs=pl.BlockSpec((TCB, D), lambda i: (i, 0)),
    out_shape=jax.ShapeDtypeStruct((P, D), _f32),
)


def _tc_prep_body(degp_ref, u0_ref, dinv_ref, u0n_lo_ref, u0n_hi_ref):
    deg = degp_ref[0][:, 0:1] + degp_ref[1][:, 0:1] + 1.0
    dv = 1.0 / jnp.sqrt(deg)
    dinv_ref[...] = dv
    u0n = dv * u0_ref[...]
    u0n_lo_ref[...] = u0n[:, :H]
    u0n_hi_ref[...] = u0n[:, H:]


_tc_prep = pl.pallas_call(
    _tc_prep_body,
    grid=(TCG,),
    in_specs=[
        pl.BlockSpec((2, TCB, W1W), lambda i: (0, i, 0)),
        pl.BlockSpec((TCB, D), lambda i: (i, 0)),
    ],
    out_specs=[
        pl.BlockSpec((TCB, 1), lambda i: (i, 0)),
        pl.BlockSpec((TCB, H), lambda i: (i, 0)),
        pl.BlockSpec((TCB, H), lambda i: (i, 0)),
    ],
    out_shape=[
        jax.ShapeDtypeStruct((P, 1), _f32),
        jax.ShapeDtypeStruct((P, H), _f32),
        jax.ShapeDtypeStruct((P, H), _f32),
    ],
)


def _tc_h0u1_body(acc0_ref, u0n_lo_ref, u0n_hi_ref, dinv_ref, b0_ref,
                  w1_ref, u1n_ref):
    dv = dinv_ref[...]
    h0 = jnp.concatenate(
        [dv * (acc0_ref[0] + u0n_lo_ref[...]),
         dv * (acc0_ref[1] + u0n_hi_ref[...])], axis=1) + b0_ref[...]
    u1 = jnp.dot(h0, w1_ref[...], preferred_element_type=_f32)
    u1n_ref[...] = dv * u1


_tc_h0u1 = pl.pallas_call(
    _tc_h0u1_body,
    grid=(TCG,),
    in_specs=[
        pl.BlockSpec((2, TCB, H), lambda i: (0, i, 0)),
        pl.BlockSpec((TCB, H), lambda i: (i, 0)),
        pl.BlockSpec((TCB, H), lambda i: (i, 0)),
        pl.BlockSpec((TCB, 1), lambda i: (i, 0)),
        pl.BlockSpec((1, D), lambda i: (0, 0)),
        pl.BlockSpec((D, H), lambda i: (0, 0)),
    ],
    out_specs=pl.BlockSpec((TCB, H), lambda i: (i, 0)),
    out_shape=jax.ShapeDtypeStruct((P, H), _f32),
)


def _tc_h1_body(acc1_ref, u1n_ref, dinv_ref, b1_ref, w2_ref, batch_ref,
                z2n16_ref, pool_ref, cnt_ref):
    i = pl.program_id(0)
    dv = dinv_ref[...]
    hpre = dv * (acc1_ref[0] + acc1_ref[1] + u1n_ref[...]) + b1_ref[...]
    h1 = jnp.where(hpre > 0, hpre, NEG_SLOPE * hpre)
    z2 = jnp.dot(h1, w2_ref[...], preferred_element_type=_f32)
    z2n16_ref[...] = (dv * z2) * jnp.ones((1, W1W), _f32)
    b = batch_ref[...]
    pool_rows, cnt_rows = [], []
    for g in range(G):
        m = (b == g).astype(_f32)
        pool_rows.append(jnp.sum(h1 * m, axis=0, keepdims=True))
        cnt_rows.append(jnp.sum(m * jnp.ones((1, H), _f32), axis=0,
                                keepdims=True))
    pool_blk = jnp.concatenate(pool_rows, axis=0)
    cnt_blk = jnp.concatenate(cnt_rows, axis=0)

    @pl.when(i == 0)
    def _():
        pool_ref[...] = pool_blk
        cnt_ref[...] = cnt_blk

    @pl.when(i > 0)
    def _():
        pool_ref[...] += pool_blk
        cnt_ref[...] += cnt_blk


_tc_h1 = pl.pallas_call(
    _tc_h1_body,
    grid=(TCG,),
    in_specs=[
        pl.BlockSpec((2, TCB, H), lambda i: (0, i, 0)),
        pl.BlockSpec((TCB, H), lambda i: (i, 0)),
        pl.BlockSpec((TCB, 1), lambda i: (i, 0)),
        pl.BlockSpec((1, H), lambda i: (0, 0)),
        pl.BlockSpec((H, 1), lambda i: (0, 0)),
        pl.BlockSpec((TCB, 1), lambda i: (i, 0)),
    ],
    out_specs=[
        pl.BlockSpec((TCB, W1W), lambda i: (i, 0)),
        pl.BlockSpec((G, H), lambda i: (0, 0)),
        pl.BlockSpec((G, H), lambda i: (0, 0)),
    ],
    out_shape=[
        jax.ShapeDtypeStruct((P, W1W), _f32),
        jax.ShapeDtypeStruct((G, H), _f32),
        jax.ShapeDtypeStruct((G, H), _f32),
    ],
)


def _tc_final_body(accz_ref, z2n_ref, dinv_ref, b2_ref, pool_ref, cnt_ref,
                   weos_ref, beos_ref, yl_ref, ye_ref):
    dv = dinv_ref[...]
    yl_ref[...] = (dv * (accz_ref[0][:, 0:1] + accz_ref[1][:, 0:1]
                         + z2n_ref[...][:, 0:1]) + b2_ref[...])

    @pl.when(pl.program_id(0) == 0)
    def _():
        xp = pool_ref[...] / jnp.maximum(cnt_ref[...], 1.0)
        ye_ref[...] = (jnp.dot(xp, weos_ref[...], preferred_element_type=_f32)
                       + beos_ref[...])


_tc_final = pl.pallas_call(
    _tc_final_body,
    grid=(TCG,),
    in_specs=[
        pl.BlockSpec((2, TCB, W1W), lambda i: (0, i, 0)),
        pl.BlockSpec((TCB, W1W), lambda i: (i, 0)),
        pl.BlockSpec((TCB, 1), lambda i: (i, 0)),
        pl.BlockSpec((1, 1), lambda i: (0, 0)),
        pl.BlockSpec((G, H), lambda i: (0, 0)),
        pl.BlockSpec((G, H), lambda i: (0, 0)),
        pl.BlockSpec((H, 1), lambda i: (0, 0)),
        pl.BlockSpec((1, 1), lambda i: (0, 0)),
    ],
    out_specs=[
        pl.BlockSpec((TCB, 1), lambda i: (i, 0)),
        pl.BlockSpec((G, 1), lambda i: (0, 0)),
    ],
    out_shape=[
        jax.ShapeDtypeStruct((P, 1), _f32),
        jax.ShapeDtypeStruct((G, 1), _f32),
    ],
)


def kernel(x, edge_index, batch, W0, b0, W1, b1, W2, b2, W_eos, b_eos):
    src = edge_index[0].astype(jnp.int32)
    dst = edge_index[1].astype(jnp.int32)
    pad = jnp.full((EP - E,), N, jnp.int32)
    srcm = jnp.concatenate([src, pad]).reshape(EP // EB, EB)
    dstm = jnp.concatenate([dst, pad]).reshape(EP // EB, EB)

    x_pad = jnp.pad(x, ((0, P - N), (0, 0)))
    batchp = jnp.pad(batch.astype(jnp.int32), (0, P - N),
                     constant_values=G).reshape(P, 1)

    zeros64 = jnp.zeros((ROWS_PER_TILE, H), _f32)
    zeros16 = jnp.zeros((ROWS_PER_TILE, W1W), _f32)
    ones_eb = jnp.ones((EB, W1W), _f32)

    b0r = b0.reshape(1, D)
    b1r = b1.reshape(1, H)
    b2r = b2.reshape(1, 1)
    beosr = b_eos.reshape(1, 1)

    u0 = _tc_u0(x_pad, W0)
    degp = _sc_deg(dstm, ones_eb, zeros16)
    dinv, u0n_lo, u0n_hi = _tc_prep(degp, u0)
    acc0 = _sc_prop128(srcm, dstm, u0n_lo, u0n_hi, zeros64)
    u1n = _tc_h0u1(acc0, u0n_lo, u0n_hi, dinv, b0r, W1)
    acc1 = _sc_prop64(srcm, dstm, u1n, zeros64)
    z2n16, pool, counts = _tc_h1(acc1, u1n, dinv, b1r, W2, batchp)
    accz = _sc_prop16(srcm, dstm, z2n16, zeros16)
    y_leafp, y_eosp = _tc_final(accz, z2n16, dinv, b2r, pool, counts,
                                W_eos, beosr)
    return y_leafp[:N, 0], y_eosp[:, 0]


# speedup vs baseline: 32.7292x; 1.8794x over previous
import functools

import jax
import jax.numpy as jnp
from jax import lax
from jax.experimental import pallas as pl
from jax.experimental.pallas import tpu as pltpu
from jax.experimental.pallas import tpu_sc as plsc

N = 10000
E = 320000
D = 128
H = 64
G = 8
NEG_SLOPE = 0.01

P = 10240
EB = 128
TILES = 32
NBLK = 80
EP = TILES * NBLK * EB
ROWS_PER_TILE = P // 16
W1W = 16
TCB = 512
TCG = P // TCB

_mesh = plsc.VectorSubcoreMesh(core_axis_name="c", subcore_axis_name="s")
_f32 = jnp.float32
_params = pltpu.CompilerParams(use_tc_tiling_on_sc=False)


def _zero_phase(s, zsrcs_dsts):
    for zsrc, dst in zsrcs_dsts:
        pltpu.sync_copy(zsrc, dst.at[pl.ds(s * ROWS_PER_TILE, ROWS_PER_TILE)])
    plsc.subcore_barrier()


def _copy_out(c, s, pairs):
    plsc.subcore_barrier()
    for sh, out in pairs:
        sl = pl.ds(s * ROWS_PER_TILE, ROWS_PER_TILE)
        pltpu.sync_copy(sh.at[sl], out.at[c, sl])


@functools.partial(
    pl.kernel,
    mesh=_mesh,
    compiler_params=_params,
    out_type=jax.ShapeDtypeStruct((2, P, W1W), _f32),
    scratch_types=[
        pltpu.VMEM((NBLK, EB), jnp.int32),
        pltpu.VMEM((EB, W1W), _f32),
        pltpu.VMEM_SHARED((P, W1W), _f32),
        pltpu.SemaphoreType.DMA,
    ],
)
def _sc_deg(dstm, ones_hbm, zeros16, deg_out, didx, onesv, accd, sem):
    c = lax.axis_index("c")
    s = lax.axis_index("s")
    wid = s * 2 + c
    pltpu.sync_copy(ones_hbm, onesv)
    pltpu.sync_copy(dstm.at[pl.ds(wid * NBLK, NBLK)], didx)
    _zero_phase(s, [(zeros16, accd)])

    @pl.loop(0, NBLK)
    def _(j):
        pltpu.async_copy(onesv, accd.at[didx.at[j]], sem, add=True)

    @pl.loop(0, NBLK)
    def _(j):
        pltpu.make_async_copy(onesv, accd.at[didx.at[j]], sem).wait()

    _copy_out(c, s, [(accd, deg_out)])


DEPTH = 5


def _prop_loop(tab, sidx, didx, rows, acc, gsems, ssems, nblk):
    for b in range(DEPTH):
        pltpu.async_copy(tab.at[sidx.at[b]], rows[b], gsems[b])

    @pl.loop(0, nblk, step=DEPTH)
    def _(j):
        descs = []
        for b in range(DEPTH):
            pltpu.make_async_copy(tab.at[sidx.at[j + b]], rows[b],
                                  gsems[b]).wait()
            descs.append(pltpu.async_copy(rows[b], acc.at[didx.at[j + b]],
                                          ssems[b], add=True))
        for b in range(DEPTH):
            @pl.when(j + b + DEPTH < nblk)
            def _(b=b):
                descs[b].wait()
                pltpu.async_copy(tab.at[sidx.at[j + b + DEPTH]], rows[b],
                                 gsems[b])

    for b in range(DEPTH):
        pltpu.make_async_copy(rows[b], acc.at[didx.at[nblk - DEPTH + b]],
                              ssems[b]).wait()


NBLK2 = 2 * NBLK


@functools.partial(
    pl.kernel,
    mesh=_mesh,
    compiler_params=_params,
    out_type=jax.ShapeDtypeStruct((2, P, H), _f32),
    scratch_types=(
        [pltpu.VMEM((NBLK2, EB), jnp.int32),
         pltpu.VMEM((NBLK2, EB), jnp.int32)]
        + [pltpu.VMEM((EB, H), _f32)] * DEPTH
        + [pltpu.VMEM_SHARED((P, H), _f32)]
        + [pltpu.SemaphoreType.DMA] * (2 * DEPTH)
    ),
)
def _sc_prop128(srcm, dstm, tab_lo, tab_hi, zeros, acc_out,
                sidx, didx, *rest):
    rows = rest[:DEPTH]
    acc = rest[DEPTH]
    gsems = rest[DEPTH + 1:2 * DEPTH + 1]
    ssems = rest[2 * DEPTH + 1:]
    c = lax.axis_index("c")
    s = lax.axis_index("s")
    pltpu.sync_copy(srcm.at[pl.ds(s * NBLK2, NBLK2)], sidx)
    pltpu.sync_copy(dstm.at[pl.ds(s * NBLK2, NBLK2)], didx)
    _zero_phase(s, [(zeros, acc)])

    @pl.when(c == 0)
    def _():
        _prop_loop(tab_lo, sidx, didx, rows, acc, gsems, ssems, NBLK2)

    @pl.when(c == 1)
    def _():
        _prop_loop(tab_hi, sidx, didx, rows, acc, gsems, ssems, NBLK2)

    _copy_out(c, s, [(acc, acc_out)])


def _make_prop(width):
    @functools.partial(
        pl.kernel,
        mesh=_mesh,
        compiler_params=_params,
        out_type=jax.ShapeDtypeStruct((2, P, width), _f32),
        scratch_types=(
            [pltpu.VMEM((NBLK, EB), jnp.int32),
             pltpu.VMEM((NBLK, EB), jnp.int32)]
            + [pltpu.VMEM((EB, width), _f32)] * DEPTH
            + [pltpu.VMEM_SHARED((P, width), _f32)]
            + [pltpu.SemaphoreType.DMA] * (2 * DEPTH)
        ),
    )
    def _prop(srcm, dstm, tab, zeros, acc_out, sidx, didx, *rest):
        rows = rest[:DEPTH]
        acc = rest[DEPTH]
        gsems = rest[DEPTH + 1:2 * DEPTH + 1]
        ssems = rest[2 * DEPTH + 1:]
        c = lax.axis_index("c")
        s = lax.axis_index("s")
        wid = s * 2 + c
        pltpu.sync_copy(srcm.at[pl.ds(wid * NBLK, NBLK)], sidx)
        pltpu.sync_copy(dstm.at[pl.ds(wid * NBLK, NBLK)], didx)
        _zero_phase(s, [(zeros, acc)])
        _prop_loop(tab, sidx, didx, rows, acc, gsems, ssems, NBLK)
        _copy_out(c, s, [(acc, acc_out)])

    return _prop


_sc_prop64 = _make_prop(H)
_sc_prop16 = _make_prop(W1W)


def _tc_u0_body(x_ref, w0_ref, u0_ref):
    u0_ref[...] = jnp.dot(x_ref[...], w0_ref[...], preferred_element_type=_f32)


_tc_u0 = pl.pallas_call(
    _tc_u0_body,
    grid=(TCG,),
    in_specs=[
        pl.BlockSpec((TCB, D), lambda i: (i, 0)),
        pl.BlockSpec((D, D), lambda i: (0, 0)),
    ],
    out_specs=pl.BlockSpec((TCB, D), lambda i: (i, 0)),
    out_shape=jax.ShapeDtypeStruct((P, D), _f32),
)


def _tc_prep_body(degp_ref, u0_ref, dinv_ref, u0n_lo_ref, u0n_hi_ref):
    deg = degp_ref[0][:, 0:1] + degp_ref[1][:, 0:1] + 1.0
    dv = 1.0 / jnp.sqrt(deg)
    dinv_ref[...] = dv
    u0n = dv * u0_ref[...]
    u0n_lo_ref[...] = u0n[:, :H]
    u0n_hi_ref[...] = u0n[:, H:]


_tc_prep = pl.pallas_call(
    _tc_prep_body,
    grid=(TCG,),
    in_specs=[
        pl.BlockSpec((2, TCB, W1W), lambda i: (0, i, 0)),
        pl.BlockSpec((TCB, D), lambda i: (i, 0)),
    ],
    out_specs=[
        pl.BlockSpec((TCB, 1), lambda i: (i, 0)),
        pl.BlockSpec((TCB, H), lambda i: (i, 0)),
        pl.BlockSpec((TCB, H), lambda i: (i, 0)),
    ],
    out_shape=[
        jax.ShapeDtypeStruct((P, 1), _f32),
        jax.ShapeDtypeStruct((P, H), _f32),
        jax.ShapeDtypeStruct((P, H), _f32),
    ],
)


def _tc_h0u1_body(acc0_ref, u0n_lo_ref, u0n_hi_ref, dinv_ref, b0_ref,
                  w1_ref, u1n_ref):
    dv = dinv_ref[...]
    h0 = jnp.concatenate(
        [dv * (acc0_ref[0] + u0n_lo_ref[...]),
         dv * (acc0_ref[1] + u0n_hi_ref[...])], axis=1) + b0_ref[...]
    u1 = jnp.dot(h0, w1_ref[...], preferred_element_type=_f32)
    u1n_ref[...] = dv * u1


_tc_h0u1 = pl.pallas_call(
    _tc_h0u1_body,
    grid=(TCG,),
    in_specs=[
        pl.BlockSpec((2, TCB, H), lambda i: (0, i, 0)),
        pl.BlockSpec((TCB, H), lambda i: (i, 0)),
        pl.BlockSpec((TCB, H), lambda i: (i, 0)),
        pl.BlockSpec((TCB, 1), lambda i: (i, 0)),
        pl.BlockSpec((1, D), lambda i: (0, 0)),
        pl.BlockSpec((D, H), lambda i: (0, 0)),
    ],
    out_specs=pl.BlockSpec((TCB, H), lambda i: (i, 0)),
    out_shape=jax.ShapeDtypeStruct((P, H), _f32),
)


def _tc_h1_body(acc1_ref, u1n_ref, dinv_ref, b1_ref, w2_ref, batch_ref,
                z2n16_ref, pool_ref, cnt_ref):
    i = pl.program_id(0)
    dv = dinv_ref[...]
    hpre = dv * (acc1_ref[0] + acc1_ref[1] + u1n_ref[...]) + b1_ref[...]
    h1 = jnp.where(hpre > 0, hpre, NEG_SLOPE * hpre)
    z2 = jnp.dot(h1, w2_ref[...], preferred_element_type=_f32)
    z2n16_ref[...] = (dv * z2) * jnp.ones((1, W1W), _f32)
    b = batch_ref[...]
    pool_rows, cnt_rows = [], []
    for g in range(G):
        m = (b == g).astype(_f32)
        pool_rows.append(jnp.sum(h1 * m, axis=0, keepdims=True))
        cnt_rows.append(jnp.sum(m * jnp.ones((1, H), _f32), axis=0,
                                keepdims=True))
    pool_blk = jnp.concatenate(pool_rows, axis=0)
    cnt_blk = jnp.concatenate(cnt_rows, axis=0)

    @pl.when(i == 0)
    def _():
        pool_ref[...] = pool_blk
        cnt_ref[...] = cnt_blk

    @pl.when(i > 0)
    def _():
        pool_ref[...] += pool_blk
        cnt_ref[...] += cnt_blk


_tc_h1 = pl.pallas_call(
    _tc_h1_body,
    grid=(TCG,),
    in_specs=[
        pl.BlockSpec((2, TCB, H), lambda i: (0, i, 0)),
        pl.BlockSpec((TCB, H), lambda i: (i, 0)),
        pl.BlockSpec((TCB, 1), lambda i: (i, 0)),
        pl.BlockSpec((1, H), lambda i: (0, 0)),
        pl.BlockSpec((H, 1), lambda i: (0, 0)),
        pl.BlockSpec((TCB, 1), lambda i: (i, 0)),
    ],
    out_specs=[
        pl.BlockSpec((TCB, W1W), lambda i: (i, 0)),
        pl.BlockSpec((G, H), lambda i: (0, 0)),
        pl.BlockSpec((G, H), lambda i: (0, 0)),
    ],
    out_shape=[
        jax.ShapeDtypeStruct((P, W1W), _f32),
        jax.ShapeDtypeStruct((G, H), _f32),
        jax.ShapeDtypeStruct((G, H), _f32),
    ],
)


def _tc_final_body(accz_ref, z2n_ref, dinv_ref, b2_ref, pool_ref, cnt_ref,
                   weos_ref, beos_ref, yl_ref, ye_ref):
    dv = dinv_ref[...]
    yl_ref[...] = (dv * (accz_ref[0][:, 0:1] + accz_ref[1][:, 0:1]
                         + z2n_ref[...][:, 0:1]) + b2_ref[...])

    @pl.when(pl.program_id(0) == 0)
    def _():
        xp = pool_ref[...] / jnp.maximum(cnt_ref[...], 1.0)
        ye_ref[...] = (jnp.dot(xp, weos_ref[...], preferred_element_type=_f32)
                       + beos_ref[...])


_tc_final = pl.pallas_call(
    _tc_final_body,
    grid=(TCG,),
    in_specs=[
        pl.BlockSpec((2, TCB, W1W), lambda i: (0, i, 0)),
        pl.BlockSpec((TCB, W1W), lambda i: (i, 0)),
        pl.BlockSpec((TCB, 1), lambda i: (i, 0)),
        pl.BlockSpec((1, 1), lambda i: (0, 0)),
        pl.BlockSpec((G, H), lambda i: (0, 0)),
        pl.BlockSpec((G, H), lambda i: (0, 0)),
        pl.BlockSpec((H, 1), lambda i: (0, 0)),
        pl.BlockSpec((1, 1), lambda i: (0, 0)),
    ],
    out_specs=[
        pl.BlockSpec((TCB, 1), lambda i: (i, 0)),
        pl.BlockSpec((G, 1), lambda i: (0, 0)),
    ],
    out_shape=[
        jax.ShapeDtypeStruct((P, 1), _f32),
        jax.ShapeDtypeStruct((G, 1), _f32),
    ],
)


def kernel(x, edge_index, batch, W0, b0, W1, b1, W2, b2, W_eos, b_eos):
    src = edge_index[0].astype(jnp.int32)
    dst = edge_index[1].astype(jnp.int32)
    pad = N + jnp.arange(EP - E, dtype=jnp.int32) % (P - N)
    srcm = jnp.concatenate([src, pad]).reshape(EP // EB, EB)
    dstm = jnp.concatenate([dst, pad]).reshape(EP // EB, EB)

    x_pad = jnp.pad(x, ((0, P - N), (0, 0)))
    batchp = jnp.pad(batch.astype(jnp.int32), (0, P - N),
                     constant_values=G).reshape(P, 1)

    zeros64 = jnp.zeros((ROWS_PER_TILE, H), _f32)
    zeros16 = jnp.zeros((ROWS_PER_TILE, W1W), _f32)
    ones_eb = jnp.ones((EB, W1W), _f32)

    b0r = b0.reshape(1, D)
    b1r = b1.reshape(1, H)
    b2r = b2.reshape(1, 1)
    beosr = b_eos.reshape(1, 1)

    u0 = _tc_u0(x_pad, W0)
    degp = _sc_deg(dstm, ones_eb, zeros16)
    dinv, u0n_lo, u0n_hi = _tc_prep(degp, u0)
    acc0 = _sc_prop128(srcm, dstm, u0n_lo, u0n_hi, zeros64)
    u1n = _tc_h0u1(acc0, u0n_lo, u0n_hi, dinv, b0r, W1)
    acc1 = _sc_prop64(srcm, dstm, u1n, zeros64)
    z2n16, pool, counts = _tc_h1(acc1, u1n, dinv, b1r, W2, batchp)
    accz = _sc_prop16(srcm, dstm, z2n16, zeros16)
    y_leafp, y_eosp = _tc_final(accz, z2n16, dinv, b2r, pool, counts,
                                W_eos, beosr)
    return y_leafp[:N, 0], y_eosp[:, 0]


# speedup vs baseline: 33.5831x; 1.0261x over previous
import functools

import jax
import jax.numpy as jnp
from jax import lax
from jax.experimental import pallas as pl
from jax.experimental.pallas import tpu as pltpu
from jax.experimental.pallas import tpu_sc as plsc

N = 10000
E = 320000
D = 128
H = 64
G = 8
NEG_SLOPE = 0.01

P = 10240
EB = 128
TILES = 32
NBLK = 80
EP = TILES * NBLK * EB
ROWS_PER_TILE = P // 16
W1W = 16
TCB = 512
TCG = P // TCB

_mesh = plsc.VectorSubcoreMesh(core_axis_name="c", subcore_axis_name="s")
_f32 = jnp.float32
_params = pltpu.CompilerParams(use_tc_tiling_on_sc=False)


def _zero_phase(s, zsrcs_dsts):
    for zsrc, dst in zsrcs_dsts:
        pltpu.sync_copy(zsrc, dst.at[pl.ds(s * ROWS_PER_TILE, ROWS_PER_TILE)])
    plsc.subcore_barrier()


def _copy_out(c, s, pairs):
    plsc.subcore_barrier()
    for sh, out in pairs:
        sl = pl.ds(s * ROWS_PER_TILE, ROWS_PER_TILE)
        pltpu.sync_copy(sh.at[sl], out.at[c, sl])


@functools.partial(
    pl.kernel,
    mesh=_mesh,
    compiler_params=_params,
    out_type=jax.ShapeDtypeStruct((2, P, W1W), _f32),
    scratch_types=[
        pltpu.VMEM((NBLK, EB), jnp.int32),
        pltpu.VMEM((EB, W1W), _f32),
        pltpu.VMEM_SHARED((P, W1W), _f32),
        pltpu.SemaphoreType.DMA,
    ],
)
def _sc_deg(dstm, ones_hbm, zeros16, deg_out, didx, onesv, accd, sem):
    c = lax.axis_index("c")
    s = lax.axis_index("s")
    wid = s * 2 + c
    pltpu.sync_copy(ones_hbm, onesv)
    pltpu.sync_copy(dstm.at[pl.ds(wid * NBLK, NBLK)], didx)
    _zero_phase(s, [(zeros16, accd)])

    @pl.loop(0, NBLK)
    def _(j):
        pltpu.async_copy(onesv, accd.at[didx.at[j]], sem, add=True)

    @pl.loop(0, NBLK)
    def _(j):
        pltpu.make_async_copy(onesv, accd.at[didx.at[j]], sem).wait()

    _copy_out(c, s, [(accd, deg_out)])


DEPTH128 = 5
DEPTH = 8


def _prop_loop(tab, sidx, didx, rows, acc, gsems, ssems, nblk, depth):
    for b in range(depth):
        pltpu.async_copy(tab.at[sidx.at[b]], rows[b], gsems[b])

    @pl.loop(0, nblk, step=depth)
    def _(j):
        descs = []
        for b in range(depth):
            pltpu.make_async_copy(tab.at[sidx.at[j + b]], rows[b],
                                  gsems[b]).wait()
            descs.append(pltpu.async_copy(rows[b], acc.at[didx.at[j + b]],
                                          ssems[b], add=True))
        for b in range(depth):
            @pl.when(j + b + depth < nblk)
            def _(b=b):
                descs[b].wait()
                pltpu.async_copy(tab.at[sidx.at[j + b + depth]], rows[b],
                                 gsems[b])

    for b in range(depth):
        pltpu.make_async_copy(rows[b], acc.at[didx.at[nblk - depth + b]],
                              ssems[b]).wait()


NBLK2 = 2 * NBLK


@functools.partial(
    pl.kernel,
    mesh=_mesh,
    compiler_params=_params,
    out_type=jax.ShapeDtypeStruct((2, P, H), _f32),
    scratch_types=(
        [pltpu.VMEM((NBLK2, EB), jnp.int32),
         pltpu.VMEM((NBLK2, EB), jnp.int32)]
        + [pltpu.VMEM((EB, H), _f32)] * DEPTH128
        + [pltpu.VMEM_SHARED((P, H), _f32)]
        + [pltpu.SemaphoreType.DMA] * (2 * DEPTH128)
    ),
)
def _sc_prop128(srcm, dstm, tab_lo, tab_hi, zeros, acc_out,
                sidx, didx, *rest):
    rows = rest[:DEPTH128]
    acc = rest[DEPTH128]
    gsems = rest[DEPTH128 + 1:2 * DEPTH128 + 1]
    ssems = rest[2 * DEPTH128 + 1:]
    c = lax.axis_index("c")
    s = lax.axis_index("s")
    pltpu.sync_copy(srcm.at[pl.ds(s * NBLK2, NBLK2)], sidx)
    pltpu.sync_copy(dstm.at[pl.ds(s * NBLK2, NBLK2)], didx)
    _zero_phase(s, [(zeros, acc)])

    @pl.when(c == 0)
    def _():
        _prop_loop(tab_lo, sidx, didx, rows, acc, gsems, ssems, NBLK2,
                   DEPTH128)

    @pl.when(c == 1)
    def _():
        _prop_loop(tab_hi, sidx, didx, rows, acc, gsems, ssems, NBLK2,
                   DEPTH128)

    _copy_out(c, s, [(acc, acc_out)])


def _make_prop(width):
    @functools.partial(
        pl.kernel,
        mesh=_mesh,
        compiler_params=_params,
        out_type=jax.ShapeDtypeStruct((2, P, width), _f32),
        scratch_types=(
            [pltpu.VMEM((NBLK, EB), jnp.int32),
             pltpu.VMEM((NBLK, EB), jnp.int32)]
            + [pltpu.VMEM((EB, width), _f32)] * DEPTH
            + [pltpu.VMEM_SHARED((P, width), _f32)]
            + [pltpu.SemaphoreType.DMA] * (2 * DEPTH)
        ),
    )
    def _prop(srcm, dstm, tab, zeros, acc_out, sidx, didx, *rest):
        rows = rest[:DEPTH]
        acc = rest[DEPTH]
        gsems = rest[DEPTH + 1:2 * DEPTH + 1]
        ssems = rest[2 * DEPTH + 1:]
        c = lax.axis_index("c")
        s = lax.axis_index("s")
        wid = s * 2 + c
        pltpu.sync_copy(srcm.at[pl.ds(wid * NBLK, NBLK)], sidx)
        pltpu.sync_copy(dstm.at[pl.ds(wid * NBLK, NBLK)], didx)
        _zero_phase(s, [(zeros, acc)])
        _prop_loop(tab, sidx, didx, rows, acc, gsems, ssems, NBLK, DEPTH)
        _copy_out(c, s, [(acc, acc_out)])

    return _prop


_sc_prop64 = _make_prop(H)
_sc_prop16 = _make_prop(W1W)


def _tc_prep_body(degp_ref, x_ref, w0_ref, dinv_ref, u0n_lo_ref, u0n_hi_ref):
    deg = degp_ref[0][:, 0:1] + degp_ref[1][:, 0:1] + 1.0
    dv = 1.0 / jnp.sqrt(deg)
    dinv_ref[...] = dv
    u0 = jnp.dot(x_ref[...], w0_ref[...], preferred_element_type=_f32)
    u0n = dv * u0
    u0n_lo_ref[...] = u0n[:, :H]
    u0n_hi_ref[...] = u0n[:, H:]


_tc_prep = pl.pallas_call(
    _tc_prep_body,
    grid=(TCG,),
    in_specs=[
        pl.BlockSpec((2, TCB, W1W), lambda i: (0, i, 0)),
        pl.BlockSpec((TCB, D), lambda i: (i, 0)),
        pl.BlockSpec((D, D), lambda i: (0, 0)),
    ],
    out_specs=[
        pl.BlockSpec((TCB, 1), lambda i: (i, 0)),
        pl.BlockSpec((TCB, H), lambda i: (i, 0)),
        pl.BlockSpec((TCB, H), lambda i: (i, 0)),
    ],
    out_shape=[
        jax.ShapeDtypeStruct((P, 1), _f32),
        jax.ShapeDtypeStruct((P, H), _f32),
        jax.ShapeDtypeStruct((P, H), _f32),
    ],
)


def _tc_h0u1_body(acc0_ref, u0n_lo_ref, u0n_hi_ref, dinv_ref, b0_ref,
                  w1_ref, u1n_ref):
    dv = dinv_ref[...]
    h0 = jnp.concatenate(
        [dv * (acc0_ref[0] + u0n_lo_ref[...]),
         dv * (acc0_ref[1] + u0n_hi_ref[...])], axis=1) + b0_ref[...]
    u1 = jnp.dot(h0, w1_ref[...], preferred_element_type=_f32)
    u1n_ref[...] = dv * u1


_tc_h0u1 = pl.pallas_call(
    _tc_h0u1_body,
    grid=(TCG,),
    in_specs=[
        pl.BlockSpec((2, TCB, H), lambda i: (0, i, 0)),
        pl.BlockSpec((TCB, H), lambda i: (i, 0)),
        pl.BlockSpec((TCB, H), lambda i: (i, 0)),
        pl.BlockSpec((TCB, 1), lambda i: (i, 0)),
        pl.BlockSpec((1, D), lambda i: (0, 0)),
        pl.BlockSpec((D, H), lambda i: (0, 0)),
    ],
    out_specs=pl.BlockSpec((TCB, H), lambda i: (i, 0)),
    out_shape=jax.ShapeDtypeStruct((P, H), _f32),
)


def _tc_h1_body(acc1_ref, u1n_ref, dinv_ref, b1_ref, w2_ref, batch_ref,
                z2n16_ref, pool_ref, cnt_ref):
    i = pl.program_id(0)
    dv = dinv_ref[...]
    hpre = dv * (acc1_ref[0] + acc1_ref[1] + u1n_ref[...]) + b1_ref[...]
    h1 = jnp.where(hpre > 0, hpre, NEG_SLOPE * hpre)
    z2 = jnp.dot(h1, w2_ref[...], preferred_element_type=_f32)
    z2n16_ref[...] = (dv * z2) * jnp.ones((1, W1W), _f32)
    b = batch_ref[...]
    pool_rows, cnt_rows = [], []
    for g in range(G):
        m = (b == g).astype(_f32)
        pool_rows.append(jnp.sum(h1 * m, axis=0, keepdims=True))
        cnt_rows.append(jnp.sum(m * jnp.ones((1, H), _f32), axis=0,
                                keepdims=True))
    pool_blk = jnp.concatenate(pool_rows, axis=0)
    cnt_blk = jnp.concatenate(cnt_rows, axis=0)

    @pl.when(i == 0)
    def _():
        pool_ref[...] = pool_blk
        cnt_ref[...] = cnt_blk

    @pl.when(i > 0)
    def _():
        pool_ref[...] += pool_blk
        cnt_ref[...] += cnt_blk


_tc_h1 = pl.pallas_call(
    _tc_h1_body,
    grid=(TCG,),
    in_specs=[
        pl.BlockSpec((2, TCB, H), lambda i: (0, i, 0)),
        pl.BlockSpec((TCB, H), lambda i: (i, 0)),
        pl.BlockSpec((TCB, 1), lambda i: (i, 0)),
        pl.BlockSpec((1, H), lambda i: (0, 0)),
        pl.BlockSpec((H, 1), lambda i: (0, 0)),
        pl.BlockSpec((TCB, 1), lambda i: (i, 0)),
    ],
    out_specs=[
        pl.BlockSpec((TCB, W1W), lambda i: (i, 0)),
        pl.BlockSpec((G, H), lambda i: (0, 0)),
        pl.BlockSpec((G, H), lambda i: (0, 0)),
    ],
    out_shape=[
        jax.ShapeDtypeStruct((P, W1W), _f32),
        jax.ShapeDtypeStruct((G, H), _f32),
        jax.ShapeDtypeStruct((G, H), _f32),
    ],
)


def _tc_final_body(accz_ref, z2n_ref, dinv_ref, b2_ref, pool_ref, cnt_ref,
                   weos_ref, beos_ref, yl_ref, ye_ref):
    dv = dinv_ref[...]
    yl_ref[...] = (dv * (accz_ref[0][:, 0:1] + accz_ref[1][:, 0:1]
                         + z2n_ref[...][:, 0:1]) + b2_ref[...])

    @pl.when(pl.program_id(0) == 0)
    def _():
        xp = pool_ref[...] / jnp.maximum(cnt_ref[...], 1.0)
        ye_ref[...] = (jnp.dot(xp, weos_ref[...], preferred_element_type=_f32)
                       + beos_ref[...])


_tc_final = pl.pallas_call(
    _tc_final_body,
    grid=(TCG,),
    in_specs=[
        pl.BlockSpec((2, TCB, W1W), lambda i: (0, i, 0)),
        pl.BlockSpec((TCB, W1W), lambda i: (i, 0)),
        pl.BlockSpec((TCB, 1), lambda i: (i, 0)),
        pl.BlockSpec((1, 1), lambda i: (0, 0)),
        pl.BlockSpec((G, H), lambda i: (0, 0)),
        pl.BlockSpec((G, H), lambda i: (0, 0)),
        pl.BlockSpec((H, 1), lambda i: (0, 0)),
        pl.BlockSpec((1, 1), lambda i: (0, 0)),
    ],
    out_specs=[
        pl.BlockSpec((TCB, 1), lambda i: (i, 0)),
        pl.BlockSpec((G, 1), lambda i: (0, 0)),
    ],
    out_shape=[
        jax.ShapeDtypeStruct((P, 1), _f32),
        jax.ShapeDtypeStruct((G, 1), _f32),
    ],
)


def kernel(x, edge_index, batch, W0, b0, W1, b1, W2, b2, W_eos, b_eos):
    src = edge_index[0].astype(jnp.int32)
    dst = edge_index[1].astype(jnp.int32)
    pad = N + jnp.arange(EP - E, dtype=jnp.int32) % (P - N)
    srcm = jnp.concatenate([src, pad]).reshape(EP // EB, EB)
    dstm = jnp.concatenate([dst, pad]).reshape(EP // EB, EB)

    x_pad = jnp.pad(x, ((0, P - N), (0, 0)))
    batchp = jnp.pad(batch.astype(jnp.int32), (0, P - N),
                     constant_values=G).reshape(P, 1)

    zeros64 = jnp.zeros((ROWS_PER_TILE, H), _f32)
    zeros16 = jnp.zeros((ROWS_PER_TILE, W1W), _f32)
    ones_eb = jnp.ones((EB, W1W), _f32)

    b0r = b0.reshape(1, D)
    b1r = b1.reshape(1, H)
    b2r = b2.reshape(1, 1)
    beosr = b_eos.reshape(1, 1)

    degp = _sc_deg(dstm, ones_eb, zeros16)
    dinv, u0n_lo, u0n_hi = _tc_prep(degp, x_pad, W0)
    acc0 = _sc_prop128(srcm, dstm, u0n_lo, u0n_hi, zeros64)
    u1n = _tc_h0u1(acc0, u0n_lo, u0n_hi, dinv, b0r, W1)
    acc1 = _sc_prop64(srcm, dstm, u1n, zeros64)
    z2n16, pool, counts = _tc_h1(acc1, u1n, dinv, b1r, W2, batchp)
    accz = _sc_prop16(srcm, dstm, z2n16, zeros16)
    y_leafp, y_eosp = _tc_final(accz, z2n16, dinv, b2r, pool, counts,
                                W_eos, beosr)
    return y_leafp[:N, 0], y_eosp[:, 0]


# speedup vs baseline: 36.3574x; 1.0826x over previous
import functools

import jax
import jax.numpy as jnp
from jax import lax
from jax.experimental import pallas as pl
from jax.experimental.pallas import tpu as pltpu
from jax.experimental.pallas import tpu_sc as plsc

N = 10000
E = 320000
D = 128
H = 64
G = 8
NEG_SLOPE = 0.01

P = 10240
EB = 128
TILES = 32
NBLK = 80
EP = TILES * NBLK * EB
ROWS_PER_TILE = P // 16
W1W = 16
TCB = 2048
TCG = P // TCB

_mesh = plsc.VectorSubcoreMesh(core_axis_name="c", subcore_axis_name="s")
_f32 = jnp.float32
_params = pltpu.CompilerParams(use_tc_tiling_on_sc=False)


def _zero_phase(s, zsrcs_dsts):
    for zsrc, dst in zsrcs_dsts:
        pltpu.sync_copy(zsrc, dst.at[pl.ds(s * ROWS_PER_TILE, ROWS_PER_TILE)])
    plsc.subcore_barrier()


def _copy_out(c, s, pairs):
    plsc.subcore_barrier()
    for sh, out in pairs:
        sl = pl.ds(s * ROWS_PER_TILE, ROWS_PER_TILE)
        pltpu.sync_copy(sh.at[sl], out.at[c, sl])


@functools.partial(
    pl.kernel,
    mesh=_mesh,
    compiler_params=_params,
    out_type=jax.ShapeDtypeStruct((2, P, W1W), _f32),
    scratch_types=[
        pltpu.VMEM((NBLK, EB), jnp.int32),
        pltpu.VMEM((EB, W1W), _f32),
        pltpu.VMEM_SHARED((P, W1W), _f32),
        pltpu.SemaphoreType.DMA,
    ],
)
def _sc_deg(dstm, ones_hbm, zeros16, deg_out, didx, onesv, accd, sem):
    c = lax.axis_index("c")
    s = lax.axis_index("s")
    wid = s * 2 + c
    pltpu.sync_copy(ones_hbm, onesv)
    pltpu.sync_copy(dstm.at[pl.ds(wid * NBLK, NBLK)], didx)
    _zero_phase(s, [(zeros16, accd)])

    @pl.loop(0, NBLK)
    def _(j):
        pltpu.async_copy(onesv, accd.at[didx.at[j]], sem, add=True)

    @pl.loop(0, NBLK)
    def _(j):
        pltpu.make_async_copy(onesv, accd.at[didx.at[j]], sem).wait()

    _copy_out(c, s, [(accd, deg_out)])


DEPTH128 = 5
DEPTH = 8


def _prop_loop(tab, sidx, didx, rows, acc, gsems, ssems, nblk, depth):
    for b in range(depth):
        pltpu.async_copy(tab.at[sidx.at[b]], rows[b], gsems[b])

    @pl.loop(0, nblk, step=depth)
    def _(j):
        descs = []
        for b in range(depth):
            pltpu.make_async_copy(tab.at[sidx.at[j + b]], rows[b],
                                  gsems[b]).wait()
            descs.append(pltpu.async_copy(rows[b], acc.at[didx.at[j + b]],
                                          ssems[b], add=True))
        for b in range(depth):
            @pl.when(j + b + depth < nblk)
            def _(b=b):
                descs[b].wait()
                pltpu.async_copy(tab.at[sidx.at[j + b + depth]], rows[b],
                                 gsems[b])

    for b in range(depth):
        pltpu.make_async_copy(rows[b], acc.at[didx.at[nblk - depth + b]],
                              ssems[b]).wait()


NBLK2 = 2 * NBLK


@functools.partial(
    pl.kernel,
    mesh=_mesh,
    compiler_params=_params,
    out_type=jax.ShapeDtypeStruct((2, P, H), _f32),
    scratch_types=(
        [pltpu.VMEM((NBLK2, EB), jnp.int32),
         pltpu.VMEM((NBLK2, EB), jnp.int32)]
        + [pltpu.VMEM((EB, H), _f32)] * DEPTH128
        + [pltpu.VMEM_SHARED((P, H), _f32)]
        + [pltpu.SemaphoreType.DMA] * (2 * DEPTH128)
    ),
)
def _sc_prop128(srcm, dstm, tab_lo, tab_hi, zeros, acc_out,
                sidx, didx, *rest):
    rows = rest[:DEPTH128]
    acc = rest[DEPTH128]
    gsems = rest[DEPTH128 + 1:2 * DEPTH128 + 1]
    ssems = rest[2 * DEPTH128 + 1:]
    c = lax.axis_index("c")
    s = lax.axis_index("s")
    pltpu.sync_copy(srcm.at[pl.ds(s * NBLK2, NBLK2)], sidx)
    pltpu.sync_copy(dstm.at[pl.ds(s * NBLK2, NBLK2)], didx)
    _zero_phase(s, [(zeros, acc)])

    @pl.when(c == 0)
    def _():
        _prop_loop(tab_lo, sidx, didx, rows, acc, gsems, ssems, NBLK2,
                   DEPTH128)

    @pl.when(c == 1)
    def _():
        _prop_loop(tab_hi, sidx, didx, rows, acc, gsems, ssems, NBLK2,
                   DEPTH128)

    _copy_out(c, s, [(acc, acc_out)])


def _make_prop(width):
    @functools.partial(
        pl.kernel,
        mesh=_mesh,
        compiler_params=_params,
        out_type=jax.ShapeDtypeStruct((2, P, width), _f32),
        scratch_types=(
            [pltpu.VMEM((NBLK, EB), jnp.int32),
             pltpu.VMEM((NBLK, EB), jnp.int32)]
            + [pltpu.VMEM((EB, width), _f32)] * DEPTH
            + [pltpu.VMEM_SHARED((P, width), _f32)]
            + [pltpu.SemaphoreType.DMA] * (2 * DEPTH)
        ),
    )
    def _prop(srcm, dstm, tab, zeros, acc_out, sidx, didx, *rest):
        rows = rest[:DEPTH]
        acc = rest[DEPTH]
        gsems = rest[DEPTH + 1:2 * DEPTH + 1]
        ssems = rest[2 * DEPTH + 1:]
        c = lax.axis_index("c")
        s = lax.axis_index("s")
        wid = s * 2 + c
        pltpu.sync_copy(srcm.at[pl.ds(wid * NBLK, NBLK)], sidx)
        pltpu.sync_copy(dstm.at[pl.ds(wid * NBLK, NBLK)], didx)
        _zero_phase(s, [(zeros, acc)])
        _prop_loop(tab, sidx, didx, rows, acc, gsems, ssems, NBLK, DEPTH)
        _copy_out(c, s, [(acc, acc_out)])

    return _prop


_sc_prop64 = _make_prop(H)
_sc_prop16 = _make_prop(W1W)


def _tc_prep_body(degp_ref, x_ref, w0_ref, dinv_ref, u0n_lo_ref, u0n_hi_ref):
    deg = degp_ref[0][:, 0:1] + degp_ref[1][:, 0:1] + 1.0
    dv = 1.0 / jnp.sqrt(deg)
    dinv_ref[...] = dv
    u0 = jnp.dot(x_ref[...], w0_ref[...], preferred_element_type=_f32)
    u0n = dv * u0
    u0n_lo_ref[...] = u0n[:, :H]
    u0n_hi_ref[...] = u0n[:, H:]


_tc_prep = pl.pallas_call(
    _tc_prep_body,
    grid=(TCG,),
    in_specs=[
        pl.BlockSpec((2, TCB, W1W), lambda i: (0, i, 0)),
        pl.BlockSpec((TCB, D), lambda i: (i, 0)),
        pl.BlockSpec((D, D), lambda i: (0, 0)),
    ],
    out_specs=[
        pl.BlockSpec((TCB, 1), lambda i: (i, 0)),
        pl.BlockSpec((TCB, H), lambda i: (i, 0)),
        pl.BlockSpec((TCB, H), lambda i: (i, 0)),
    ],
    out_shape=[
        jax.ShapeDtypeStruct((P, 1), _f32),
        jax.ShapeDtypeStruct((P, H), _f32),
        jax.ShapeDtypeStruct((P, H), _f32),
    ],
)


def _tc_h0u1_body(acc0_ref, u0n_lo_ref, u0n_hi_ref, dinv_ref, b0_ref,
                  w1_ref, u1n_ref):
    dv = dinv_ref[...]
    h0 = jnp.concatenate(
        [dv * (acc0_ref[0] + u0n_lo_ref[...]),
         dv * (acc0_ref[1] + u0n_hi_ref[...])], axis=1) + b0_ref[...]
    u1 = jnp.dot(h0, w1_ref[...], preferred_element_type=_f32)
    u1n_ref[...] = dv * u1


_tc_h0u1 = pl.pallas_call(
    _tc_h0u1_body,
    grid=(TCG,),
    in_specs=[
        pl.BlockSpec((2, TCB, H), lambda i: (0, i, 0)),
        pl.BlockSpec((TCB, H), lambda i: (i, 0)),
        pl.BlockSpec((TCB, H), lambda i: (i, 0)),
        pl.BlockSpec((TCB, 1), lambda i: (i, 0)),
        pl.BlockSpec((1, D), lambda i: (0, 0)),
        pl.BlockSpec((D, H), lambda i: (0, 0)),
    ],
    out_specs=pl.BlockSpec((TCB, H), lambda i: (i, 0)),
    out_shape=jax.ShapeDtypeStruct((P, H), _f32),
)


def _tc_h1_body(acc1_ref, u1n_ref, dinv_ref, b1_ref, w2_ref, batch_ref,
                z2n16_ref, pool_ref, cnt_ref):
    i = pl.program_id(0)
    dv = dinv_ref[...]
    hpre = dv * (acc1_ref[0] + acc1_ref[1] + u1n_ref[...]) + b1_ref[...]
    h1 = jnp.where(hpre > 0, hpre, NEG_SLOPE * hpre)
    z2 = jnp.dot(h1, w2_ref[...], preferred_element_type=_f32)
    z2n16_ref[...] = (dv * z2) * jnp.ones((1, W1W), _f32)
    b = batch_ref[...]
    pool_rows, cnt_rows = [], []
    for g in range(G):
        m = (b == g).astype(_f32)
        pool_rows.append(jnp.sum(h1 * m, axis=0, keepdims=True))
        cnt_rows.append(jnp.sum(m * jnp.ones((1, H), _f32), axis=0,
                                keepdims=True))
    pool_blk = jnp.concatenate(pool_rows, axis=0)
    cnt_blk = jnp.concatenate(cnt_rows, axis=0)

    @pl.when(i == 0)
    def _():
        pool_ref[...] = pool_blk
        cnt_ref[...] = cnt_blk

    @pl.when(i > 0)
    def _():
        pool_ref[...] += pool_blk
        cnt_ref[...] += cnt_blk


_tc_h1 = pl.pallas_call(
    _tc_h1_body,
    grid=(TCG,),
    in_specs=[
        pl.BlockSpec((2, TCB, H), lambda i: (0, i, 0)),
        pl.BlockSpec((TCB, H), lambda i: (i, 0)),
        pl.BlockSpec((TCB, 1), lambda i: (i, 0)),
        pl.BlockSpec((1, H), lambda i: (0, 0)),
        pl.BlockSpec((H, 1), lambda i: (0, 0)),
        pl.BlockSpec((TCB, 1), lambda i: (i, 0)),
    ],
    out_specs=[
        pl.BlockSpec((TCB, W1W), lambda i: (i, 0)),
        pl.BlockSpec((G, H), lambda i: (0, 0)),
        pl.BlockSpec((G, H), lambda i: (0, 0)),
    ],
    out_shape=[
        jax.ShapeDtypeStruct((P, W1W), _f32),
        jax.ShapeDtypeStruct((G, H), _f32),
        jax.ShapeDtypeStruct((G, H), _f32),
    ],
)


def _tc_final_body(accz_ref, z2n_ref, dinv_ref, b2_ref, pool_ref, cnt_ref,
                   weos_ref, beos_ref, yl_ref, ye_ref):
    dv = dinv_ref[...]
    yl_ref[...] = (dv * (accz_ref[0][:, 0:1] + accz_ref[1][:, 0:1]
                         + z2n_ref[...][:, 0:1]) + b2_ref[...])

    @pl.when(pl.program_id(0) == 0)
    def _():
        xp = pool_ref[...] / jnp.maximum(cnt_ref[...], 1.0)
        ye_ref[...] = (jnp.dot(xp, weos_ref[...], preferred_element_type=_f32)
                       + beos_ref[...])


_tc_final = pl.pallas_call(
    _tc_final_body,
    grid=(TCG,),
    in_specs=[
        pl.BlockSpec((2, TCB, W1W), lambda i: (0, i, 0)),
        pl.BlockSpec((TCB, W1W), lambda i: (i, 0)),
        pl.BlockSpec((TCB, 1), lambda i: (i, 0)),
        pl.BlockSpec((1, 1), lambda i: (0, 0)),
        pl.BlockSpec((G, H), lambda i: (0, 0)),
        pl.BlockSpec((G, H), lambda i: (0, 0)),
        pl.BlockSpec((H, 1), lambda i: (0, 0)),
        pl.BlockSpec((1, 1), lambda i: (0, 0)),
    ],
    out_specs=[
        pl.BlockSpec((TCB, 1), lambda i: (i, 0)),
        pl.BlockSpec((G, 1), lambda i: (0, 0)),
    ],
    out_shape=[
        jax.ShapeDtypeStruct((P, 1), _f32),
        jax.ShapeDtypeStruct((G, 1), _f32),
    ],
)


def kernel(x, edge_index, batch, W0, b0, W1, b1, W2, b2, W_eos, b_eos):
    src = edge_index[0].astype(jnp.int32)
    dst = edge_index[1].astype(jnp.int32)
    pad = N + jnp.arange(EP - E, dtype=jnp.int32) % (P - N)
    srcm = jnp.concatenate([src, pad]).reshape(EP // EB, EB)
    dstm = jnp.concatenate([dst, pad]).reshape(EP // EB, EB)

    x_pad = jnp.pad(x, ((0, P - N), (0, 0)))
    batchp = jnp.pad(batch.astype(jnp.int32), (0, P - N),
                     constant_values=G).reshape(P, 1)

    zeros64 = jnp.zeros((ROWS_PER_TILE, H), _f32)
    zeros16 = jnp.zeros((ROWS_PER_TILE, W1W), _f32)
    ones_eb = jnp.ones((EB, W1W), _f32)

    b0r = b0.reshape(1, D)
    b1r = b1.reshape(1, H)
    b2r = b2.reshape(1, 1)
    beosr = b_eos.reshape(1, 1)

    degp = _sc_deg(dstm, ones_eb, zeros16)
    dinv, u0n_lo, u0n_hi = _tc_prep(degp, x_pad, W0)
    acc0 = _sc_prop128(srcm, dstm, u0n_lo, u0n_hi, zeros64)
    u1n = _tc_h0u1(acc0, u0n_lo, u0n_hi, dinv, b0r, W1)
    acc1 = _sc_prop64(srcm, dstm, u1n, zeros64)
    z2n16, pool, counts = _tc_h1(acc1, u1n, dinv, b1r, W2, batchp)
    accz = _sc_prop16(srcm, dstm, z2n16, zeros16)
    y_leafp, y_eosp = _tc_final(accz, z2n16, dinv, b2r, pool, counts,
                                W_eos, beosr)
    return y_leafp[:N, 0], y_eosp[:, 0]
